# baseline (speedup 1.0000x reference)
import functools

import jax
import jax.numpy as jnp
import numpy as np
from jax import lax
from jax.experimental import pallas as pl
from jax.experimental.pallas import tpu as pltpu
from jax.experimental.pallas import tpu_sc as plsc

F32 = jnp.float32

B, C, H, W = 2, 128, 100, 100
N, K = 10000, 16
M = B * N
CW = C // 2

NC, NS = 2, 16
NW = NC * NS
GN = 8
GPW = 80
BP = NS * GPW * GN
M_PAD = B * BP
SPW = BP // NS
NPH = 4
RPH = NPH * K
HPW = 160
HSG = 8
CH = HSG * NPH
LANES = 16

TC_BLK = 2048

SIGMA = np.arange(C)


def _tc1_body(x_ref, cp_ref, wfs_ref, wr1s_ref, wr1_ref, wr2_ref, b_ref,
              zp_ref, p1_ref, p2_ref):
    cp = cp_ref[...]
    p1s = jnp.dot(cp, wr1s_ref[...], preferred_element_type=F32)
    z = jnp.dot(x_ref[...], wfs_ref[...], preferred_element_type=F32)
    zp_ref[...] = z + b_ref[...] + p1s
    p1_ref[...] = jnp.dot(cp, wr1_ref[...], preferred_element_type=F32)
    p2_ref[...] = jnp.dot(cp, wr2_ref[...], preferred_element_type=F32)


def _tc1(x, cp, wfs, wr1s, wr1, wr2, b):
    grid = (M_PAD // TC_BLK,)
    blk = lambda i: (i, 0)
    zero = lambda i: (0, 0)
    return pl.pallas_call(
        _tc1_body,
        grid=grid,
        in_specs=[
            pl.BlockSpec((TC_BLK, C), blk),
            pl.BlockSpec((TC_BLK, 8), blk),
            pl.BlockSpec((C, C), zero),
            pl.BlockSpec((8, C), zero),
            pl.BlockSpec((8, C), zero),
            pl.BlockSpec((8, C), zero),
            pl.BlockSpec((1, C), zero),
        ],
        out_specs=[
            pl.BlockSpec((TC_BLK, C), blk),
            pl.BlockSpec((TC_BLK, C), blk),
            pl.BlockSpec((TC_BLK, C), blk),
        ],
        out_shape=[
            jax.ShapeDtypeStruct((M_PAD, C), F32),
            jax.ShapeDtypeStruct((M_PAD, C), F32),
            jax.ShapeDtypeStruct((M_PAD, C), F32),
        ],
    )(x, cp, wfs, wr1s, wr1, wr2, b)


def _tc2_body(x_ref, cp_ref, wfs_ref, wr2s_ref, b_ref, zp_ref):
    p2s = jnp.dot(cp_ref[...], wr2s_ref[...], preferred_element_type=F32)
    z = jnp.dot(x_ref[...], wfs_ref[...], preferred_element_type=F32)
    zp_ref[...] = z + b_ref[...] + p2s


def _tc2(x, cp, wfs, wr2s, b):
    grid = (M_PAD // TC_BLK,)
    blk = lambda i: (i, 0)
    zero = lambda i: (0, 0)
    return pl.pallas_call(
        _tc2_body,
        grid=grid,
        in_specs=[
            pl.BlockSpec((TC_BLK, C), blk),
            pl.BlockSpec((TC_BLK, 8), blk),
            pl.BlockSpec((C, C), zero),
            pl.BlockSpec((8, C), zero),
            pl.BlockSpec((1, C), zero),
        ],
        out_specs=pl.BlockSpec((TC_BLK, C), blk),
        out_shape=jax.ShapeDtypeStruct((M_PAD, C), F32),
    )(x, cp, wfs, wr2s, b)


def _sc_body(z_hbm, p_hbm, x_hbm, gidx_hbm, out_hbm,
             z_sp, idx_v, rows_a, rows_b, p_ch, x_ch, sem_a, sem_b):
    cid = lax.axis_index("c")
    sid = lax.axis_index("s")
    wid = cid * NS + sid
    nbase = wid * GPW * GN

    def drain(dst, sem):
        pltpu.make_async_copy(x_hbm.at[pl.ds(0, dst.shape[0])], dst,
                              sem).wait()

    sb = pl.multiple_of(cid * BP + sid * SPW, GN)
    pltpu.sync_copy(z_hbm.at[pl.ds(sb, SPW)],
                    z_sp.at[pl.ds(pl.multiple_of(sid * SPW, GN), SPW)])
    pltpu.sync_copy(gidx_hbm.at[wid], idx_v)
    plsc.subcore_barrier()
    pltpu.async_copy(z_sp.at[idx_v.at[0]], rows_a, sem_a)

    def compute(h, rows):
        r0 = lax.rem(h, HSG) * NPH

        def node(n, carry):
            r = r0 + n
            for c in range(C // LANES):
                sl = pl.ds(c * LANES, LANES)
                pv = p_ch[r, sl]
                a = x_ch[r, sl]
                for k in range(K):
                    a = a + jnp.maximum(rows[n * K + k, sl] - pv, 0.0)
                x_ch[r, sl] = a
            return carry

        lax.fori_loop(0, NPH, node, 0)

    def outer(t, carry):
        h0 = 2 * t
        h1 = h0 + 1

        @pl.when(lax.rem(h0, HSG) == 0)
        def _load_chunk():
            cb = pl.multiple_of(nbase + (h0 // HSG) * CH, GN)
            pltpu.sync_copy(p_hbm.at[pl.ds(cb, CH)], p_ch)
            pltpu.sync_copy(x_hbm.at[pl.ds(cb, CH)], x_ch)

        pltpu.async_copy(z_sp.at[idx_v.at[h1]], rows_b, sem_b)
        drain(rows_a, sem_a)
        compute(h0, rows_a)

        @pl.when(h0 + 2 < HPW)
        def _next_a():
            pltpu.async_copy(z_sp.at[idx_v.at[h0 + 2]], rows_a, sem_a)

        drain(rows_b, sem_b)
        compute(h1, rows_b)

        @pl.when(lax.rem(h1, HSG) == HSG - 1)
        def _store_chunk():
            cb = pl.multiple_of(nbase + (h1 // HSG) * CH, GN)
            pltpu.sync_copy(x_ch, out_hbm.at[pl.ds(cb, CH)])

        return carry

    lax.fori_loop(0, HPW // 2, outer, 0)


_sc_layer = functools.partial(
    pl.kernel,
    mesh=plsc.VectorSubcoreMesh(core_axis_name="c", subcore_axis_name="s"),
    out_type=jax.ShapeDtypeStruct((M_PAD, C), F32),
    scratch_types=[
        pltpu.VMEM_SHARED((BP, C), F32),
        pltpu.VMEM((HPW, RPH), jnp.int32),
        pltpu.VMEM((RPH, C), F32),
        pltpu.VMEM((RPH, C), F32),
        pltpu.VMEM((CH, C), F32),
        pltpu.VMEM((CH, C), F32),
        pltpu.SemaphoreType.DMA,
        pltpu.SemaphoreType.DMA,
    ],
)(_sc_body)


@jax.jit
def kernel(mask, feats, coors, indices, W1, b1, W2, b2):
    x0 = jnp.transpose(feats, (0, 2, 3, 1)).reshape(B, H * W, C)
    x0 = jnp.where(mask.reshape(B, -1)[..., None], x0, 0.0)
    x0 = jnp.pad(x0, ((0, 0), (0, BP - N), (0, 0))).reshape(M_PAD, C)

    cp = jnp.concatenate([coors, jnp.zeros((B, N, 5), F32)], axis=-1)
    cp = jnp.pad(cp, ((0, 0), (0, BP - N), (0, 0))).reshape(M_PAD, 8)
    zpad = jnp.zeros((5, C), F32)
    wr1 = jnp.concatenate([W1[C:], zpad], axis=0)
    wr2 = jnp.concatenate([W2[C:], zpad], axis=0)
    wf1s = W1[:C][:, SIGMA]
    wf2s = W2[:C][:, SIGMA]
    wr1s = wr1[:, SIGMA]
    wr2s = wr2[:, SIGMA]
    b1s = b1[SIGMA].reshape(1, C)
    b2s = b2[SIGMA].reshape(1, C)

    idx_flat = indices.astype(jnp.int32).reshape(B, N * K)
    pad_idx = (jnp.arange((BP - N) * K, dtype=jnp.int32) * 157) % N
    gidx = jnp.concatenate(
        [idx_flat[0], pad_idx, idx_flat[1], pad_idx])
    gidx = gidx.reshape(NW, HPW, RPH)

    zb1, p1, p2 = _tc1(x0, cp, wf1s, wr1s, wr1, wr2, b1s)
    out1 = _sc_layer(zb1, p1, x0, gidx)
    zb2 = _tc2(out1, cp, wf2s, wr2s, b2s)
    out2 = _sc_layer(zb2, p2, out1, gidx)
    return out2.reshape(B, BP, C)[:, :N]


# speedup vs baseline: 1.1388x; 1.1388x over previous
import functools

import jax
import jax.numpy as jnp
import numpy as np
from jax import lax
from jax.experimental import pallas as pl
from jax.experimental.pallas import tpu as pltpu
from jax.experimental.pallas import tpu_sc as plsc

F32 = jnp.float32

B, C, H, W = 2, 128, 100, 100
N, K = 10000, 16
M = B * N
CW = C // 2

NC, NS = 2, 16
NW = NC * NS
GN = 8
RPG = GN * K
GPW = 80
M_PAD = NW * GPW * GN
SG = 8
CH = SG * GN
LANES = 16

TC_BLK = 2048

SIGMA = np.arange(C)


def _tc1_body(x_ref, cp_ref, wfs_ref, wr1s_ref, wr1_ref, wr2_ref, b_ref,
              zp_ref, p1_ref, p2_ref):
    cp = cp_ref[...]
    p1s = jnp.dot(cp, wr1s_ref[...], preferred_element_type=F32)
    z = jnp.dot(x_ref[...], wfs_ref[...], preferred_element_type=F32)
    zp_ref[...] = z + b_ref[...] + p1s
    p1_ref[...] = jnp.dot(cp, wr1_ref[...], preferred_element_type=F32)
    p2_ref[...] = jnp.dot(cp, wr2_ref[...], preferred_element_type=F32)


def _tc1(x, cp, wfs, wr1s, wr1, wr2, b):
    grid = (M_PAD // TC_BLK,)
    blk = lambda i: (i, 0)
    zero = lambda i: (0, 0)
    return pl.pallas_call(
        _tc1_body,
        grid=grid,
        in_specs=[
            pl.BlockSpec((TC_BLK, C), blk),
            pl.BlockSpec((TC_BLK, 8), blk),
            pl.BlockSpec((C, C), zero),
            pl.BlockSpec((8, C), zero),
            pl.BlockSpec((8, C), zero),
            pl.BlockSpec((8, C), zero),
            pl.BlockSpec((1, C), zero),
        ],
        out_specs=[
            pl.BlockSpec((TC_BLK, C), blk),
            pl.BlockSpec((TC_BLK, C), blk),
            pl.BlockSpec((TC_BLK, C), blk),
        ],
        out_shape=[
            jax.ShapeDtypeStruct((M_PAD, C), F32),
            jax.ShapeDtypeStruct((M_PAD, C), F32),
            jax.ShapeDtypeStruct((M_PAD, C), F32),
        ],
    )(x, cp, wfs, wr1s, wr1, wr2, b)


def _tc2_body(x_ref, cp_ref, wfs_ref, wr2s_ref, b_ref, zp_ref):
    p2s = jnp.dot(cp_ref[...], wr2s_ref[...], preferred_element_type=F32)
    z = jnp.dot(x_ref[...], wfs_ref[...], preferred_element_type=F32)
    zp_ref[...] = z + b_ref[...] + p2s


def _tc2(x, cp, wfs, wr2s, b):
    grid = (M_PAD // TC_BLK,)
    blk = lambda i: (i, 0)
    zero = lambda i: (0, 0)
    return pl.pallas_call(
        _tc2_body,
        grid=grid,
        in_specs=[
            pl.BlockSpec((TC_BLK, C), blk),
            pl.BlockSpec((TC_BLK, 8), blk),
            pl.BlockSpec((C, C), zero),
            pl.BlockSpec((8, C), zero),
            pl.BlockSpec((1, C), zero),
        ],
        out_specs=pl.BlockSpec((TC_BLK, C), blk),
        out_shape=jax.ShapeDtypeStruct((M_PAD, C), F32),
    )(x, cp, wfs, wr2s, b)


def _sc_body(z_hbm, p_hbm, x_hbm, gidx_hbm, out_hbm,
             idx_v, rows_a, rows_b, p_cha, p_chb, x_cha, x_chb,
             sem_a, sem_b, psem_a, psem_b, xsem_a, xsem_b, osem_a, osem_b):
    wid = lax.axis_index("s") * NC + lax.axis_index("c")
    nbase = wid * GPW * GN

    def drain(dst, sem):
        pltpu.make_async_copy(z_hbm.at[pl.ds(0, dst.shape[0])], dst,
                              sem).wait()

    n_real = (M * K - (NW - 1) * GPW * RPG) // RPG

    @pl.when(wid < NW - 1)
    def _stage_full():
        pltpu.sync_copy(gidx_hbm.at[pl.ds(wid * GPW, GPW)], idx_v)

    @pl.when(wid == NW - 1)
    def _stage_tail():
        def fill(j, carry):
            base = lax.iota(jnp.int32, LANES) * 613 + j * 131
            for c8 in range(RPG // LANES):
                sl = pl.ds(c8 * LANES, LANES)
                idx_v[j, sl] = lax.rem(base + c8 * 17, N)
            return carry

        lax.fori_loop(n_real, GPW, fill, 0)
        pltpu.sync_copy(gidx_hbm.at[pl.ds((NW - 1) * GPW, n_real)],
                        idx_v.at[pl.ds(0, n_real)])

    def addoff(j, carry):
        off = jnp.where(nbase + j * GN >= N, N, 0).astype(jnp.int32)
        for c8 in range(RPG // LANES):
            sl = pl.ds(c8 * LANES, LANES)
            idx_v[j, sl] = idx_v[j, sl] + off
        return carry

    lax.fori_loop(0, GPW, addoff, 0)

    pltpu.sync_copy(p_hbm.at[pl.ds(pl.multiple_of(nbase, GN), CH)], p_cha)
    pltpu.sync_copy(x_hbm.at[pl.ds(pl.multiple_of(nbase, GN), CH)], x_cha)
    pltpu.async_copy(z_hbm.at[idx_v.at[0]], rows_a, sem_a)

    def compute(j, rows, p_ch, x_ch):
        r0 = lax.rem(j, SG) * GN

        def node(n, carry):
            r = r0 + n
            for c in range(C // LANES):
                sl = pl.ds(c * LANES, LANES)
                pv = p_ch[r, sl]
                a = x_ch[r, sl]
                for k in range(K):
                    a = a + jnp.maximum(rows[n * K + k, sl] - pv, 0.0)
                x_ch[r, sl] = a
            return carry

        lax.fori_loop(0, GN, node, 0)

    def make_pair(jbase, p_ch, x_ch):
        def pair(t, carry):
            j0 = jbase + 2 * t
            j1 = j0 + 1
            pltpu.async_copy(z_hbm.at[idx_v.at[j1]], rows_b, sem_b)
            drain(rows_a, sem_a)
            compute(j0, rows_a, p_ch, x_ch)

            @pl.when(j0 + 2 < GPW)
            def _next_a():
                pltpu.async_copy(z_hbm.at[idx_v.at[j0 + 2]], rows_a, sem_a)

            drain(rows_b, sem_b)
            compute(j1, rows_b, p_ch, x_ch)
            return carry

        return pair

    def outer(u, carry):
        cb_a = pl.multiple_of(nbase + u * 2 * CH, GN)
        cb_b = pl.multiple_of(cb_a + CH, GN)

        @pl.when(u > 0)
        def _wait_b_store():
            drain(x_chb, osem_b)

        pltpu.async_copy(p_hbm.at[pl.ds(cb_b, CH)], p_chb, psem_b)
        pltpu.async_copy(x_hbm.at[pl.ds(cb_b, CH)], x_chb, xsem_b)

        lax.fori_loop(0, SG // 2, make_pair(u * 2 * SG, p_cha, x_cha), 0)
        pltpu.async_copy(x_cha, out_hbm.at[pl.ds(cb_a, CH)], osem_a)

        @pl.when(u + 1 < GPW // (2 * SG))
        def _prefetch_a():
            drain(x_cha, osem_a)
            cb_n = pl.multiple_of(nbase + (u + 1) * 2 * CH, GN)
            pltpu.async_copy(p_hbm.at[pl.ds(cb_n, CH)], p_cha, psem_a)
            pltpu.async_copy(x_hbm.at[pl.ds(cb_n, CH)], x_cha, xsem_a)

        drain(p_chb, psem_b)
        drain(x_chb, xsem_b)
        lax.fori_loop(0, SG // 2, make_pair(u * 2 * SG + SG, p_chb, x_chb), 0)
        pltpu.async_copy(x_chb, out_hbm.at[pl.ds(cb_b, CH)], osem_b)

        @pl.when(u + 1 < GPW // (2 * SG))
        def _wait_a_load():
            drain(p_cha, psem_a)
            drain(x_cha, xsem_a)

        return carry

    lax.fori_loop(0, GPW // (2 * SG), outer, 0)
    drain(x_cha, osem_a)
    drain(x_chb, osem_b)


_sc_layer = functools.partial(
    pl.kernel,
    mesh=plsc.VectorSubcoreMesh(core_axis_name="c", subcore_axis_name="s"),
    out_type=jax.ShapeDtypeStruct((M_PAD, C), F32),
    scratch_types=[
        pltpu.VMEM((GPW, RPG), jnp.int32),
        pltpu.VMEM((RPG, C), F32),
        pltpu.VMEM((RPG, C), F32),
        pltpu.VMEM((CH, C), F32),
        pltpu.VMEM((CH, C), F32),
        pltpu.VMEM((CH, C), F32),
        pltpu.VMEM((CH, C), F32),
        pltpu.SemaphoreType.DMA,
        pltpu.SemaphoreType.DMA,
        pltpu.SemaphoreType.DMA,
        pltpu.SemaphoreType.DMA,
        pltpu.SemaphoreType.DMA,
        pltpu.SemaphoreType.DMA,
        pltpu.SemaphoreType.DMA,
        pltpu.SemaphoreType.DMA,
    ],
)(_sc_body)


@jax.jit
def kernel(mask, feats, coors, indices, W1, b1, W2, b2):
    x0 = jnp.transpose(feats, (0, 2, 3, 1)).reshape(B, H * W, C)
    x0 = jnp.where(mask.reshape(B, -1)[..., None], x0, 0.0).reshape(M, C)
    x0 = jnp.pad(x0, ((0, M_PAD - M), (0, 0)))

    cp = jnp.concatenate([coors, jnp.zeros((B, N, 5), F32)], axis=-1)
    cp = jnp.pad(cp.reshape(M, 8), ((0, M_PAD - M), (0, 0)))
    zpad = jnp.zeros((5, C), F32)
    wr1 = jnp.concatenate([W1[C:], zpad], axis=0)
    wr2 = jnp.concatenate([W2[C:], zpad], axis=0)
    wf1s = W1[:C][:, SIGMA]
    wf2s = W2[:C][:, SIGMA]
    wr1s = wr1[:, SIGMA]
    wr2s = wr2[:, SIGMA]
    b1s = b1[SIGMA].reshape(1, C)
    b2s = b2[SIGMA].reshape(1, C)

    gidx = indices.astype(jnp.int32).reshape(M * K // RPG, RPG)

    zb1, p1, p2 = _tc1(x0, cp, wf1s, wr1s, wr1, wr2, b1s)
    out1 = _sc_layer(zb1, p1, x0, gidx)
    zb2 = _tc2(out1, cp, wf2s, wr2s, b2s)
    out2 = _sc_layer(zb2, p2, out1, gidx)
    return out2[:M].reshape(B, N, C)


# speedup vs baseline: 1.1404x; 1.0014x over previous
import functools

import jax
import jax.numpy as jnp
import numpy as np
from jax import lax
from jax.experimental import pallas as pl
from jax.experimental.pallas import tpu as pltpu
from jax.experimental.pallas import tpu_sc as plsc

F32 = jnp.float32

B, C, H, W = 2, 128, 100, 100
N, K = 10000, 16
M = B * N
CW = C // 2

NC, NS = 2, 16
NW = NC * NS
GN = 8
RPG = GN * K
GPW = 80
M_PAD = NW * GPW * GN
SG = 8
CH = SG * GN
LANES = 16

TC_BLK = 2048

SIGMA = np.arange(C)


def _tc1_body(x_ref, cp_ref, wfs_ref, wr1s_ref, wr1_ref, wr2_ref, b_ref,
              zp_ref, p1_ref, p2_ref):
    cp = cp_ref[...]
    p1s = jnp.dot(cp, wr1s_ref[...], preferred_element_type=F32)
    z = jnp.dot(x_ref[...], wfs_ref[...], preferred_element_type=F32)
    zp_ref[...] = z + b_ref[...] + p1s
    p1_ref[...] = jnp.dot(cp, wr1_ref[...], preferred_element_type=F32)
    p2_ref[...] = jnp.dot(cp, wr2_ref[...], preferred_element_type=F32)


def _tc1(x, cp, wfs, wr1s, wr1, wr2, b):
    grid = (M_PAD // TC_BLK,)
    blk = lambda i: (i, 0)
    zero = lambda i: (0, 0)
    return pl.pallas_call(
        _tc1_body,
        grid=grid,
        in_specs=[
            pl.BlockSpec((TC_BLK, C), blk),
            pl.BlockSpec((TC_BLK, 8), blk),
            pl.BlockSpec((C, C), zero),
            pl.BlockSpec((8, C), zero),
            pl.BlockSpec((8, C), zero),
            pl.BlockSpec((8, C), zero),
            pl.BlockSpec((1, C), zero),
        ],
        out_specs=[
            pl.BlockSpec((TC_BLK, C), blk),
            pl.BlockSpec((TC_BLK, C), blk),
            pl.BlockSpec((TC_BLK, C), blk),
        ],
        out_shape=[
            jax.ShapeDtypeStruct((M_PAD, C), F32),
            jax.ShapeDtypeStruct((M_PAD, C), F32),
            jax.ShapeDtypeStruct((M_PAD, C), F32),
        ],
    )(x, cp, wfs, wr1s, wr1, wr2, b)


def _tc2_body(x_ref, cp_ref, wfs_ref, wr2s_ref, b_ref, zp_ref):
    p2s = jnp.dot(cp_ref[...], wr2s_ref[...], preferred_element_type=F32)
    z = jnp.dot(x_ref[...], wfs_ref[...], preferred_element_type=F32)
    zp_ref[...] = z + b_ref[...] + p2s


def _tc2(x, cp, wfs, wr2s, b):
    grid = (M_PAD // TC_BLK,)
    blk = lambda i: (i, 0)
    zero = lambda i: (0, 0)
    return pl.pallas_call(
        _tc2_body,
        grid=grid,
        in_specs=[
            pl.BlockSpec((TC_BLK, C), blk),
            pl.BlockSpec((TC_BLK, 8), blk),
            pl.BlockSpec((C, C), zero),
            pl.BlockSpec((8, C), zero),
            pl.BlockSpec((1, C), zero),
        ],
        out_specs=pl.BlockSpec((TC_BLK, C), blk),
        out_shape=jax.ShapeDtypeStruct((M_PAD, C), F32),
    )(x, cp, wfs, wr2s, b)


def _sc_body(z_hbm, p_hbm, x_hbm, gidx_hbm, out_hbm,
             idx_v, rows_a, rows_b, rows_c, rows_d,
             p_cha, p_chb, x_cha, x_chb,
             sem_a, sem_b, sem_c, sem_d,
             psem_a, psem_b, xsem_a, xsem_b, osem_a, osem_b):
    wid = lax.axis_index("s") * NC + lax.axis_index("c")
    nbase = wid * GPW * GN

    def drain(dst, sem):
        pltpu.make_async_copy(z_hbm.at[pl.ds(0, dst.shape[0])], dst,
                              sem).wait()

    n_real = (M * K - (NW - 1) * GPW * RPG) // RPG

    @pl.when(wid < NW - 1)
    def _stage_full():
        pltpu.sync_copy(gidx_hbm.at[pl.ds(wid * GPW, GPW)], idx_v)

    @pl.when(wid == NW - 1)
    def _stage_tail():
        def fill(j, carry):
            base = lax.iota(jnp.int32, LANES) * 613 + j * 131
            for c8 in range(RPG // LANES):
                sl = pl.ds(c8 * LANES, LANES)
                idx_v[j, sl] = lax.rem(base + c8 * 17, N)
            return carry

        lax.fori_loop(n_real, GPW, fill, 0)
        pltpu.sync_copy(gidx_hbm.at[pl.ds((NW - 1) * GPW, n_real)],
                        idx_v.at[pl.ds(0, n_real)])

    def addoff(j, carry):
        off = jnp.where(nbase + j * GN >= N, N, 0).astype(jnp.int32)
        for c8 in range(RPG // LANES):
            sl = pl.ds(c8 * LANES, LANES)
            idx_v[j, sl] = idx_v[j, sl] + off
        return carry

    lax.fori_loop(0, GPW, addoff, 0)

    pltpu.sync_copy(p_hbm.at[pl.ds(pl.multiple_of(nbase, GN), CH)], p_cha)
    pltpu.sync_copy(x_hbm.at[pl.ds(pl.multiple_of(nbase, GN), CH)], x_cha)
    pltpu.async_copy(z_hbm.at[idx_v.at[0]], rows_a, sem_a)
    pltpu.async_copy(z_hbm.at[idx_v.at[1]], rows_b, sem_b)
    pltpu.async_copy(z_hbm.at[idx_v.at[2]], rows_c, sem_c)

    def compute(j, rows, p_ch, x_ch):
        r0 = lax.rem(j, SG) * GN

        def node(n, carry):
            r = r0 + n
            for c in range(C // LANES):
                sl = pl.ds(c * LANES, LANES)
                pv = p_ch[r, sl]
                a = x_ch[r, sl]
                for k in range(K):
                    a = a + jnp.maximum(rows[n * K + k, sl] - pv, 0.0)
                x_ch[r, sl] = a
            return carry

        lax.fori_loop(0, GN, node, 0)

    def make_quad(jbase, p_ch, x_ch):
        bufs = ((rows_a, sem_a), (rows_b, sem_b),
                (rows_c, sem_c), (rows_d, sem_d))

        def quad(t, carry):
            j = jbase + 4 * t
            pltpu.async_copy(z_hbm.at[idx_v.at[j + 3]], rows_d, sem_d)
            for i in range(4):
                rows_i, sem_i = bufs[i]
                drain(rows_i, sem_i)
                compute(j + i, rows_i, p_ch, x_ch)
                if i < 3:
                    nxt = j + 4 + i

                    @pl.when(nxt < GPW)
                    def _next():
                        pltpu.async_copy(z_hbm.at[idx_v.at[nxt]],
                                         bufs[i][0], bufs[i][1])
            return carry

        return quad

    def outer(u, carry):
        cb_a = pl.multiple_of(nbase + u * 2 * CH, GN)
        cb_b = pl.multiple_of(cb_a + CH, GN)

        @pl.when(u > 0)
        def _wait_b_store():
            drain(x_chb, osem_b)

        pltpu.async_copy(p_hbm.at[pl.ds(cb_b, CH)], p_chb, psem_b)
        pltpu.async_copy(x_hbm.at[pl.ds(cb_b, CH)], x_chb, xsem_b)

        lax.fori_loop(0, SG // 4, make_quad(u * 2 * SG, p_cha, x_cha), 0)
        pltpu.async_copy(x_cha, out_hbm.at[pl.ds(cb_a, CH)], osem_a)

        @pl.when(u + 1 < GPW // (2 * SG))
        def _prefetch_a():
            drain(x_cha, osem_a)
            cb_n = pl.multiple_of(nbase + (u + 1) * 2 * CH, GN)
            pltpu.async_copy(p_hbm.at[pl.ds(cb_n, CH)], p_cha, psem_a)
            pltpu.async_copy(x_hbm.at[pl.ds(cb_n, CH)], x_cha, xsem_a)

        drain(p_chb, psem_b)
        drain(x_chb, xsem_b)
        lax.fori_loop(0, SG // 4, make_quad(u * 2 * SG + SG, p_chb, x_chb), 0)
        pltpu.async_copy(x_chb, out_hbm.at[pl.ds(cb_b, CH)], osem_b)

        @pl.when(u + 1 < GPW // (2 * SG))
        def _wait_a_load():
            drain(p_cha, psem_a)
            drain(x_cha, xsem_a)

        return carry

    lax.fori_loop(0, GPW // (2 * SG), outer, 0)
    drain(x_cha, osem_a)
    drain(x_chb, osem_b)


_sc_layer = functools.partial(
    pl.kernel,
    mesh=plsc.VectorSubcoreMesh(core_axis_name="c", subcore_axis_name="s"),
    out_type=jax.ShapeDtypeStruct((M_PAD, C), F32),
    scratch_types=[
        pltpu.VMEM((GPW, RPG), jnp.int32),
        pltpu.VMEM((RPG, C), F32),
        pltpu.VMEM((RPG, C), F32),
        pltpu.VMEM((RPG, C), F32),
        pltpu.VMEM((RPG, C), F32),
        pltpu.VMEM((CH, C), F32),
        pltpu.VMEM((CH, C), F32),
        pltpu.VMEM((CH, C), F32),
        pltpu.VMEM((CH, C), F32),
        pltpu.SemaphoreType.DMA,
        pltpu.SemaphoreType.DMA,
        pltpu.SemaphoreType.DMA,
        pltpu.SemaphoreType.DMA,
        pltpu.SemaphoreType.DMA,
        pltpu.SemaphoreType.DMA,
        pltpu.SemaphoreType.DMA,
        pltpu.SemaphoreType.DMA,
        pltpu.SemaphoreType.DMA,
        pltpu.SemaphoreType.DMA,
    ],
)(_sc_body)


@jax.jit
def kernel(mask, feats, coors, indices, W1, b1, W2, b2):
    x0 = jnp.transpose(feats, (0, 2, 3, 1)).reshape(B, H * W, C)
    x0 = jnp.where(mask.reshape(B, -1)[..., None], x0, 0.0).reshape(M, C)
    x0 = jnp.pad(x0, ((0, M_PAD - M), (0, 0)))

    cp = jnp.concatenate([coors, jnp.zeros((B, N, 5), F32)], axis=-1)
    cp = jnp.pad(cp.reshape(M, 8), ((0, M_PAD - M), (0, 0)))
    zpad = jnp.zeros((5, C), F32)
    wr1 = jnp.concatenate([W1[C:], zpad], axis=0)
    wr2 = jnp.concatenate([W2[C:], zpad], axis=0)
    wf1s = W1[:C][:, SIGMA]
    wf2s = W2[:C][:, SIGMA]
    wr1s = wr1[:, SIGMA]
    wr2s = wr2[:, SIGMA]
    b1s = b1[SIGMA].reshape(1, C)
    b2s = b2[SIGMA].reshape(1, C)

    gidx = indices.astype(jnp.int32).reshape(M * K // RPG, RPG)

    zb1, p1, p2 = _tc1(x0, cp, wf1s, wr1s, wr1, wr2, b1s)
    out1 = _sc_layer(zb1, p1, x0, gidx)
    zb2 = _tc2(out1, cp, wf2s, wr2s, b2s)
    out2 = _sc_layer(zb2, p2, out1, gidx)
    return out2[:M].reshape(B, N, C)


# speedup vs baseline: 1.1436x; 1.0028x over previous
import functools

import jax
import jax.numpy as jnp
import numpy as np
from jax import lax
from jax.experimental import pallas as pl
from jax.experimental.pallas import tpu as pltpu
from jax.experimental.pallas import tpu_sc as plsc

F32 = jnp.float32

B, C, H, W = 2, 128, 100, 100
N, K = 10000, 16
M = B * N
CW = C // 2

NC, NS = 2, 16
NW = NC * NS
GN = 8
RPG = GN * K
GPW = 80
M_PAD = NW * GPW * GN
SG = 8
CH = SG * GN
LANES = 16

TC_BLK = 2048

SIGMA = np.arange(C)


def _tc1_body(x_ref, cp_ref, wfs_ref, wr1s_ref, wr1_ref, wr2_ref, b_ref,
              zp_ref, p1_ref, p2_ref):
    cp = cp_ref[...]
    p1s = jnp.dot(cp, wr1s_ref[...], preferred_element_type=F32)
    z = jnp.dot(x_ref[...], wfs_ref[...], preferred_element_type=F32)
    zp_ref[...] = z + b_ref[...] + p1s
    p1_ref[...] = jnp.dot(cp, wr1_ref[...], preferred_element_type=F32)
    p2_ref[...] = jnp.dot(cp, wr2_ref[...], preferred_element_type=F32)


def _tc1(x, cp, wfs, wr1s, wr1, wr2, b):
    grid = (M_PAD // TC_BLK,)
    blk = lambda i: (i, 0)
    zero = lambda i: (0, 0)
    return pl.pallas_call(
        _tc1_body,
        grid=grid,
        in_specs=[
            pl.BlockSpec((TC_BLK, C), blk),
            pl.BlockSpec((TC_BLK, 8), blk),
            pl.BlockSpec((C, C), zero),
            pl.BlockSpec((8, C), zero),
            pl.BlockSpec((8, C), zero),
            pl.BlockSpec((8, C), zero),
            pl.BlockSpec((1, C), zero),
        ],
        out_specs=[
            pl.BlockSpec((TC_BLK, C), blk),
            pl.BlockSpec((TC_BLK, C), blk),
            pl.BlockSpec((TC_BLK, C), blk),
        ],
        out_shape=[
            jax.ShapeDtypeStruct((M_PAD, C), F32),
            jax.ShapeDtypeStruct((M_PAD, C), F32),
            jax.ShapeDtypeStruct((M_PAD, C), F32),
        ],
    )(x, cp, wfs, wr1s, wr1, wr2, b)


def _tc2_body(x_ref, cp_ref, wfs_ref, wr2s_ref, b_ref, zp_ref):
    p2s = jnp.dot(cp_ref[...], wr2s_ref[...], preferred_element_type=F32)
    z = jnp.dot(x_ref[...], wfs_ref[...], preferred_element_type=F32)
    zp_ref[...] = z + b_ref[...] + p2s


def _tc2(x, cp, wfs, wr2s, b):
    grid = (M_PAD // TC_BLK,)
    blk = lambda i: (i, 0)
    zero = lambda i: (0, 0)
    return pl.pallas_call(
        _tc2_body,
        grid=grid,
        in_specs=[
            pl.BlockSpec((TC_BLK, C), blk),
            pl.BlockSpec((TC_BLK, 8), blk),
            pl.BlockSpec((C, C), zero),
            pl.BlockSpec((8, C), zero),
            pl.BlockSpec((1, C), zero),
        ],
        out_specs=pl.BlockSpec((TC_BLK, C), blk),
        out_shape=jax.ShapeDtypeStruct((M_PAD, C), F32),
    )(x, cp, wfs, wr2s, b)


def _sc_body(z_hbm, p_hbm, x_hbm, gidx_hbm, out_hbm,
             idx_v, rows_a, rows_b, p_cha, p_chb, x_cha, x_chb,
             sem_a, sem_b, psem_a, psem_b, xsem_a, xsem_b, osem_a, osem_b):
    wid = lax.axis_index("s") * NC + lax.axis_index("c")
    nbase = wid * GPW * GN

    def drain(dst, sem):
        pltpu.make_async_copy(z_hbm.at[pl.ds(0, dst.shape[0])], dst,
                              sem).wait()

    pltpu.sync_copy(gidx_hbm.at[wid], idx_v)
    pltpu.sync_copy(p_hbm.at[pl.ds(pl.multiple_of(nbase, GN), CH)], p_cha)
    pltpu.sync_copy(x_hbm.at[pl.ds(pl.multiple_of(nbase, GN), CH)], x_cha)
    pltpu.async_copy(z_hbm.at[idx_v.at[0]], rows_a, sem_a)

    def compute(j, rows, p_ch, x_ch):
        r0 = lax.rem(j, SG) * GN

        def node(n, carry):
            r = r0 + n
            for c in range(C // LANES):
                sl = pl.ds(c * LANES, LANES)
                pv = p_ch[r, sl]
                a = x_ch[r, sl]
                for k in range(K):
                    a = a + jnp.maximum(rows[n * K + k, sl] - pv, 0.0)
                x_ch[r, sl] = a
            return carry

        lax.fori_loop(0, GN, node, 0)

    def make_pair(jbase, p_ch, x_ch):
        def pair(t, carry):
            j0 = jbase + 2 * t
            j1 = j0 + 1
            pltpu.async_copy(z_hbm.at[idx_v.at[j1]], rows_b, sem_b)
            drain(rows_a, sem_a)
            compute(j0, rows_a, p_ch, x_ch)

            @pl.when(j0 + 2 < GPW)
            def _next_a():
                pltpu.async_copy(z_hbm.at[idx_v.at[j0 + 2]], rows_a, sem_a)

            drain(rows_b, sem_b)
            compute(j1, rows_b, p_ch, x_ch)
            return carry

        return pair

    def outer(u, carry):
        cb_a = pl.multiple_of(nbase + u * 2 * CH, GN)
        cb_b = pl.multiple_of(cb_a + CH, GN)

        @pl.when(u > 0)
        def _wait_b_store():
            drain(x_chb, osem_b)

        pltpu.async_copy(p_hbm.at[pl.ds(cb_b, CH)], p_chb, psem_b)
        pltpu.async_copy(x_hbm.at[pl.ds(cb_b, CH)], x_chb, xsem_b)

        lax.fori_loop(0, SG // 2, make_pair(u * 2 * SG, p_cha, x_cha), 0)
        pltpu.async_copy(x_cha, out_hbm.at[pl.ds(cb_a, CH)], osem_a)

        @pl.when(u + 1 < GPW // (2 * SG))
        def _prefetch_a():
            drain(x_cha, osem_a)
            cb_n = pl.multiple_of(nbase + (u + 1) * 2 * CH, GN)
            pltpu.async_copy(p_hbm.at[pl.ds(cb_n, CH)], p_cha, psem_a)
            pltpu.async_copy(x_hbm.at[pl.ds(cb_n, CH)], x_cha, xsem_a)

        drain(p_chb, psem_b)
        drain(x_chb, xsem_b)
        lax.fori_loop(0, SG // 2, make_pair(u * 2 * SG + SG, p_chb, x_chb), 0)
        pltpu.async_copy(x_chb, out_hbm.at[pl.ds(cb_b, CH)], osem_b)

        @pl.when(u + 1 < GPW // (2 * SG))
        def _wait_a_load():
            drain(p_cha, psem_a)
            drain(x_cha, xsem_a)

        return carry

    lax.fori_loop(0, GPW // (2 * SG), outer, 0)
    drain(x_cha, osem_a)
    drain(x_chb, osem_b)


_sc_layer = functools.partial(
    pl.kernel,
    mesh=plsc.VectorSubcoreMesh(core_axis_name="c", subcore_axis_name="s"),
    out_type=jax.ShapeDtypeStruct((M_PAD, C), F32),
    scratch_types=[
        pltpu.VMEM((GPW, RPG), jnp.int32),
        pltpu.VMEM((RPG, C), F32),
        pltpu.VMEM((RPG, C), F32),
        pltpu.VMEM((CH, C), F32),
        pltpu.VMEM((CH, C), F32),
        pltpu.VMEM((CH, C), F32),
        pltpu.VMEM((CH, C), F32),
        pltpu.SemaphoreType.DMA,
        pltpu.SemaphoreType.DMA,
        pltpu.SemaphoreType.DMA,
        pltpu.SemaphoreType.DMA,
        pltpu.SemaphoreType.DMA,
        pltpu.SemaphoreType.DMA,
        pltpu.SemaphoreType.DMA,
        pltpu.SemaphoreType.DMA,
    ],
)(_sc_body)


@jax.jit
def kernel(mask, feats, coors, indices, W1, b1, W2, b2):
    x0 = jnp.transpose(feats, (0, 2, 3, 1)).reshape(B, H * W, C)
    x0 = jnp.where(mask.reshape(B, -1)[..., None], x0, 0.0).reshape(M, C)
    x0 = jnp.pad(x0, ((0, M_PAD - M), (0, 0)))

    cp = jnp.concatenate([coors, jnp.zeros((B, N, 5), F32)], axis=-1)
    cp = jnp.pad(cp.reshape(M, 8), ((0, M_PAD - M), (0, 0)))
    zpad = jnp.zeros((5, C), F32)
    wr1 = jnp.concatenate([W1[C:], zpad], axis=0)
    wr2 = jnp.concatenate([W2[C:], zpad], axis=0)
    wf1s = W1[:C][:, SIGMA]
    wf2s = W2[:C][:, SIGMA]
    wr1s = wr1[:, SIGMA]
    wr2s = wr2[:, SIGMA]
    b1s = b1[SIGMA].reshape(1, C)
    b2s = b2[SIGMA].reshape(1, C)

    gidx = indices.astype(jnp.int32) + (
        jnp.arange(B, dtype=jnp.int32) * N)[:, None, None]
    pad_n = NW * GPW * RPG - M * K
    pad_idx = (jnp.arange(pad_n, dtype=jnp.int32) * 157) % M
    gidx = jnp.concatenate([gidx.reshape(-1), pad_idx])
    gidx = gidx.reshape(NW, GPW, RPG)

    zb1, p1, p2 = _tc1(x0, cp, wf1s, wr1s, wr1, wr2, b1s)
    out1 = _sc_layer(zb1, p1, x0, gidx)
    zb2 = _tc2(out1, cp, wf2s, wr2s, b2s)
    out2 = _sc_layer(zb2, p2, out1, gidx)
    return out2[:M].reshape(B, N, C)


# speedup vs baseline: 1.1583x; 1.0129x over previous
import functools

import jax
import jax.numpy as jnp
from jax import lax
from jax.experimental import pallas as pl
from jax.experimental.pallas import tpu as pltpu
from jax.experimental.pallas import tpu_sc as plsc

F32 = jnp.float32

B, C, H, W = 2, 128, 100, 100
N, K = 10000, 16
M = B * N

NC, NS = 2, 16
NW = NC * NS
GN = 8
RPG = GN * K
GPW = 80
M_PAD = NW * GPW * GN
SG = 8
CH = SG * GN
LANES = 16

TC_BLK = 2048


def _tc1_body(x_ref, cp_ref, wf_ref, wr1_ref, wr2_ref, b_ref,
              z_ref, p1_ref, p2_ref):
    cp = cp_ref[...]
    p1 = jnp.dot(cp, wr1_ref[...], preferred_element_type=F32)
    p2 = jnp.dot(cp, wr2_ref[...], preferred_element_type=F32)
    z = jnp.dot(x_ref[...], wf_ref[...], preferred_element_type=F32)
    z_ref[...] = z + b_ref[...] + p1
    p1_ref[...] = p1
    p2_ref[...] = p2


def _tc1(x, cp, wf, wr1, wr2, b):
    grid = (M_PAD // TC_BLK,)
    blk = lambda i: (i, 0)
    zero = lambda i: (0, 0)
    return pl.pallas_call(
        _tc1_body,
        grid=grid,
        in_specs=[
            pl.BlockSpec((TC_BLK, C), blk),
            pl.BlockSpec((TC_BLK, 8), blk),
            pl.BlockSpec((C, C), zero),
            pl.BlockSpec((8, C), zero),
            pl.BlockSpec((8, C), zero),
            pl.BlockSpec((1, C), zero),
        ],
        out_specs=[
            pl.BlockSpec((TC_BLK, C), blk),
            pl.BlockSpec((TC_BLK, C), blk),
            pl.BlockSpec((TC_BLK, C), blk),
        ],
        out_shape=[
            jax.ShapeDtypeStruct((M_PAD, C), F32),
            jax.ShapeDtypeStruct((M_PAD, C), F32),
            jax.ShapeDtypeStruct((M_PAD, C), F32),
        ],
    )(x, cp, wf, wr1, wr2, b)


def _tc2_body(x_ref, wf_ref, b_ref, p2_ref, z_ref):
    z = jnp.dot(x_ref[...], wf_ref[...], preferred_element_type=F32)
    z_ref[...] = z + b_ref[...] + p2_ref[...]


def _tc2(x, wf, b, p2):
    grid = (M_PAD // TC_BLK,)
    blk = lambda i: (i, 0)
    zero = lambda i: (0, 0)
    return pl.pallas_call(
        _tc2_body,
        grid=grid,
        in_specs=[
            pl.BlockSpec((TC_BLK, C), blk),
            pl.BlockSpec((C, C), zero),
            pl.BlockSpec((1, C), zero),
            pl.BlockSpec((TC_BLK, C), blk),
        ],
        out_specs=pl.BlockSpec((TC_BLK, C), blk),
        out_shape=jax.ShapeDtypeStruct((M_PAD, C), F32),
    )(x, wf, b, p2)


def _sc_body(z_hbm, p_hbm, x_hbm, gidx_hbm, out_hbm,
             idx_v, rows_a, rows_b, p_cha, p_chb, x_cha, x_chb,
             sem_a, sem_b, psem_a, psem_b, xsem_a, xsem_b, osem_a, osem_b):
    wid = lax.axis_index("s") * NC + lax.axis_index("c")
    nbase = wid * GPW * GN

    def drain(dst, sem):
        pltpu.make_async_copy(z_hbm.at[pl.ds(0, dst.shape[0])], dst,
                              sem).wait()

    pltpu.sync_copy(gidx_hbm.at[wid], idx_v)
    pltpu.sync_copy(p_hbm.at[pl.ds(pl.multiple_of(nbase, GN), CH)], p_cha)
    pltpu.sync_copy(x_hbm.at[pl.ds(pl.multiple_of(nbase, GN), CH)], x_cha)
    pltpu.async_copy(z_hbm.at[idx_v.at[0]], rows_a, sem_a)

    def compute(j, rows, p_ch, x_ch):
        r0 = lax.rem(j, SG) * GN

        def node(n, carry):
            r = r0 + n
            for c in range(C // LANES):
                sl = pl.ds(c * LANES, LANES)
                pv = p_ch[r, sl]
                a = x_ch[r, sl]
                for k in range(K):
                    a = a + jnp.maximum(rows[n * K + k, sl] - pv, 0.0)
                x_ch[r, sl] = a
            return carry

        lax.fori_loop(0, GN, node, 0)

    def make_pair(jbase, p_ch, x_ch):
        def pair(t, carry):
            j0 = jbase + 2 * t
            j1 = j0 + 1
            pltpu.async_copy(z_hbm.at[idx_v.at[j1]], rows_b, sem_b)
            drain(rows_a, sem_a)
            compute(j0, rows_a, p_ch, x_ch)

            @pl.when(j0 + 2 < GPW)
            def _next_a():
                pltpu.async_copy(z_hbm.at[idx_v.at[j0 + 2]], rows_a, sem_a)

            drain(rows_b, sem_b)
            compute(j1, rows_b, p_ch, x_ch)
            return carry

        return pair

    def outer(u, carry):
        cb_a = pl.multiple_of(nbase + u * 2 * CH, GN)
        cb_b = pl.multiple_of(cb_a + CH, GN)

        @pl.when(u > 0)
        def _wait_b_store():
            drain(x_chb, osem_b)

        pltpu.async_copy(p_hbm.at[pl.ds(cb_b, CH)], p_chb, psem_b)
        pltpu.async_copy(x_hbm.at[pl.ds(cb_b, CH)], x_chb, xsem_b)

        lax.fori_loop(0, SG // 2, make_pair(u * 2 * SG, p_cha, x_cha), 0)
        pltpu.async_copy(x_cha, out_hbm.at[pl.ds(cb_a, CH)], osem_a)

        @pl.when(u + 1 < GPW // (2 * SG))
        def _prefetch_a():
            drain(x_cha, osem_a)
            cb_n = pl.multiple_of(nbase + (u + 1) * 2 * CH, GN)
            pltpu.async_copy(p_hbm.at[pl.ds(cb_n, CH)], p_cha, psem_a)
            pltpu.async_copy(x_hbm.at[pl.ds(cb_n, CH)], x_cha, xsem_a)

        drain(p_chb, psem_b)
        drain(x_chb, xsem_b)
        lax.fori_loop(0, SG // 2, make_pair(u * 2 * SG + SG, p_chb, x_chb), 0)
        pltpu.async_copy(x_chb, out_hbm.at[pl.ds(cb_b, CH)], osem_b)

        @pl.when(u + 1 < GPW // (2 * SG))
        def _wait_a_load():
            drain(p_cha, psem_a)
            drain(x_cha, xsem_a)

        return carry

    lax.fori_loop(0, GPW // (2 * SG), outer, 0)
    drain(x_cha, osem_a)
    drain(x_chb, osem_b)


_sc_layer = functools.partial(
    pl.kernel,
    mesh=plsc.VectorSubcoreMesh(core_axis_name="c", subcore_axis_name="s"),
    out_type=jax.ShapeDtypeStruct((M_PAD, C), F32),
    scratch_types=[
        pltpu.VMEM((GPW, RPG), jnp.int32),
        pltpu.VMEM((RPG, C), F32),
        pltpu.VMEM((RPG, C), F32),
        pltpu.VMEM((CH, C), F32),
        pltpu.VMEM((CH, C), F32),
        pltpu.VMEM((CH, C), F32),
        pltpu.VMEM((CH, C), F32),
        pltpu.SemaphoreType.DMA,
        pltpu.SemaphoreType.DMA,
        pltpu.SemaphoreType.DMA,
        pltpu.SemaphoreType.DMA,
        pltpu.SemaphoreType.DMA,
        pltpu.SemaphoreType.DMA,
        pltpu.SemaphoreType.DMA,
        pltpu.SemaphoreType.DMA,
    ],
)(_sc_body)


@jax.jit
def kernel(mask, feats, coors, indices, W1, b1, W2, b2):
    x0 = jnp.transpose(feats, (0, 2, 3, 1)).reshape(B, H * W, C)
    x0 = jnp.where(mask.reshape(B, -1)[..., None], x0, 0.0).reshape(M, C)
    x0 = jnp.pad(x0, ((0, M_PAD - M), (0, 0)))

    cp = jnp.concatenate([coors, jnp.zeros((B, N, 5), F32)], axis=-1)
    cp = jnp.pad(cp.reshape(M, 8), ((0, M_PAD - M), (0, 0)))
    zpad = jnp.zeros((5, C), F32)
    wr1 = jnp.concatenate([W1[C:], zpad], axis=0)
    wr2 = jnp.concatenate([W2[C:], zpad], axis=0)

    gidx = indices.astype(jnp.int32) + (
        jnp.arange(B, dtype=jnp.int32) * N)[:, None, None]
    pad_n = NW * GPW * RPG - M * K
    pad_idx = (jnp.arange(pad_n, dtype=jnp.int32) * 157) % M
    gidx = jnp.concatenate([gidx.reshape(-1), pad_idx])
    gidx = gidx.reshape(NW, GPW, RPG)

    z1, p1, p2 = _tc1(x0, cp, W1[:C], wr1, wr2, b1.reshape(1, C))
    out1 = _sc_layer(z1, p1, x0, gidx)
    z2 = _tc2(out1, W2[:C], b2.reshape(1, C), p2)
    out2 = _sc_layer(z2, p2, out1, gidx)
    return out2[:M].reshape(B, N, C)
